# baseline (speedup 1.0000x reference)
import functools

import jax
import jax.numpy as jnp
from jax.experimental import pallas as pl


def _epilogue(y, xb, w1_ref, b1_ref, w2_ref, b2_ref, o_ref, o16_ref):
    s = y + xb
    p = y * xb
    t = jax.lax.dot_general(
        s, w1_ref[...], (((1,), (1,)), ((), ())),
        preferred_element_type=jnp.float32)
    t = t + jax.lax.dot_general(
        p, w2_ref[...], (((1,), (1,)), ((), ())),
        preferred_element_type=jnp.float32)
    t = t + b1_ref[...] + b2_ref[...]
    t = jnp.where(t >= 0, t, 0.2 * t)
    o_ref[...] = t
    o16_ref[...] = t.astype(jnp.bfloat16)


def _layer1_body(a_ref, xf_ref, xb_ref, w1_ref, b1_ref, w2_ref, b2_ref,
                 o_ref, o16_ref, a16_ref):
    a16 = a_ref[...].astype(jnp.bfloat16)
    a16_ref[...] = a16
    y = jnp.dot(a16, xf_ref[...], preferred_element_type=jnp.float32)
    _epilogue(y, xb_ref[...], w1_ref, b1_ref, w2_ref, b2_ref, o_ref, o16_ref)


def _layer_bf16_body(a_ref, xf_ref, xb_ref, w1_ref, b1_ref, w2_ref, b2_ref,
                     o_ref, o16_ref):
    y = jnp.dot(a_ref[...], xf_ref[...], preferred_element_type=jnp.float32)
    _epilogue(y, xb_ref[...], w1_ref, b1_ref, w2_ref, b2_ref, o_ref, o16_ref)


@functools.partial(jax.jit, static_argnames=("bm",))
def _layer1(a_hat, x16, x_prev, w1, b1, w2, b2, bm=400):
    n, d = x_prev.shape
    m = a_hat.shape[0]
    return pl.pallas_call(
        _layer1_body,
        grid=(pl.cdiv(m, bm),),
        in_specs=[
            pl.BlockSpec((bm, n), lambda i: (i, 0)),
            pl.BlockSpec((n, d), lambda i: (0, 0)),
            pl.BlockSpec((bm, d), lambda i: (i, 0)),
            pl.BlockSpec((d, d), lambda i: (0, 0)),
            pl.BlockSpec((1, d), lambda i: (0, 0)),
            pl.BlockSpec((d, d), lambda i: (0, 0)),
            pl.BlockSpec((1, d), lambda i: (0, 0)),
        ],
        out_specs=[
            pl.BlockSpec((bm, d), lambda i: (i, 0)),
            pl.BlockSpec((bm, d), lambda i: (i, 0)),
            pl.BlockSpec((bm, n), lambda i: (i, 0)),
        ],
        out_shape=[
            jax.ShapeDtypeStruct((m, d), jnp.float32),
            jax.ShapeDtypeStruct((m, d), jnp.bfloat16),
            jax.ShapeDtypeStruct((m, n), jnp.bfloat16),
        ],
    )(a_hat, x16, x_prev, w1, b1, w2, b2)


@functools.partial(jax.jit, static_argnames=("bm",))
def _layer_bf16(a16, x16, x_prev, w1, b1, w2, b2, bm=1248):
    n, d = x_prev.shape
    m = a16.shape[0]
    return pl.pallas_call(
        _layer_bf16_body,
        grid=(pl.cdiv(m, bm),),
        in_specs=[
            pl.BlockSpec((bm, n), lambda i: (i, 0)),
            pl.BlockSpec((n, d), lambda i: (0, 0)),
            pl.BlockSpec((bm, d), lambda i: (i, 0)),
            pl.BlockSpec((d, d), lambda i: (0, 0)),
            pl.BlockSpec((1, d), lambda i: (0, 0)),
            pl.BlockSpec((d, d), lambda i: (0, 0)),
            pl.BlockSpec((1, d), lambda i: (0, 0)),
        ],
        out_specs=[
            pl.BlockSpec((bm, d), lambda i: (i, 0)),
            pl.BlockSpec((bm, d), lambda i: (i, 0)),
        ],
        out_shape=[
            jax.ShapeDtypeStruct((m, d), jnp.float32),
            jax.ShapeDtypeStruct((m, d), jnp.bfloat16),
        ],
    )(a16, x16, x_prev, w1, b1, w2, b2)


def kernel(x, A_hat, embed, W11, b11, W12, b12, W21, b21, W22, b22, W31, b31,
           W32, b32):
    x0 = jnp.take(embed, x, axis=0)
    x0_16 = x0.astype(jnp.bfloat16)
    b = [b.reshape(1, -1) for b in (b11, b12, b21, b22, b31, b32)]
    x1, x1_16, a16 = _layer1(A_hat, x0_16, x0, W11, b[0], W12, b[1])
    x2, x2_16 = _layer_bf16(a16, x1_16, x1, W21, b[2], W22, b[3])
    x3, _ = _layer_bf16(a16, x2_16, x2, W31, b[4], W32, b[5])
    return jnp.concatenate((x0, x1, x2, x3), axis=1)


# speedup vs baseline: 1.0158x; 1.0158x over previous
import functools

import jax
import jax.numpy as jnp
from jax import lax
from jax.experimental import pallas as pl
from jax.experimental.pallas import tpu as pltpu
from jax.experimental.pallas import tpu_sc as plsc


@functools.partial(jax.jit, static_argnames=("bw", "rem"))
def _sc_gather(embed, idx, bw, rem):
    n, d = embed.shape
    mesh = plsc.VectorSubcoreMesh(core_axis_name="c", subcore_axis_name="s")
    scratch = [
        pltpu.VMEM((bw,), jnp.int32),
        pltpu.VMEM((bw, d), jnp.float32),
        pltpu.SemaphoreType.DMA,
    ]
    if rem:
        scratch += [
            pltpu.VMEM((rem,), jnp.int32),
            pltpu.VMEM((rem, d), jnp.float32),
        ]

    @functools.partial(
        pl.kernel, mesh=mesh,
        out_type=jax.ShapeDtypeStruct(idx.shape + (d,), jnp.float32),
        scratch_types=scratch,
    )
    def body(table_hbm, idx_hbm, out_hbm, idx_v, rows_v, sem, *tail):
        wid = lax.axis_index("s") * 2 + lax.axis_index("c")
        base = wid * bw
        pltpu.sync_copy(idx_hbm.at[pl.ds(base, bw)], idx_v)
        pltpu.async_copy(table_hbm.at[idx_v], rows_v, sem).wait()
        pltpu.sync_copy(rows_v, out_hbm.at[pl.ds(base, bw)])
        if rem:
            idx2_v, rows2_v = tail

            @pl.when(wid == 0)
            def _():
                tbase = 32 * bw
                pltpu.sync_copy(idx_hbm.at[pl.ds(tbase, rem)], idx2_v)
                pltpu.async_copy(table_hbm.at[idx2_v], rows2_v, sem).wait()
                pltpu.sync_copy(rows2_v, out_hbm.at[pl.ds(tbase, rem)])

    return body(embed, idx)


def _epilogue(y, xb, w1_ref, b1_ref, w2_ref, b2_ref, o_ref, o16_ref):
    s = y + xb
    p = y * xb
    t = jax.lax.dot_general(
        s, w1_ref[...], (((1,), (1,)), ((), ())),
        preferred_element_type=jnp.float32)
    t = t + jax.lax.dot_general(
        p, w2_ref[...], (((1,), (1,)), ((), ())),
        preferred_element_type=jnp.float32)
    t = t + b1_ref[...] + b2_ref[...]
    t = jnp.where(t >= 0, t, 0.2 * t)
    o_ref[...] = t
    o16_ref[...] = t.astype(jnp.bfloat16)


def _layer1_body(a_ref, xf_ref, xb_ref, w1_ref, b1_ref, w2_ref, b2_ref,
                 o_ref, o16_ref, a16_ref):
    a16 = a_ref[...].astype(jnp.bfloat16)
    a16_ref[...] = a16
    y = jnp.dot(a16, xf_ref[...], preferred_element_type=jnp.float32)
    _epilogue(y, xb_ref[...], w1_ref, b1_ref, w2_ref, b2_ref, o_ref, o16_ref)


def _layer_bf16_body(a_ref, xf_ref, xb_ref, w1_ref, b1_ref, w2_ref, b2_ref,
                     o_ref, o16_ref):
    y = jnp.dot(a_ref[...], xf_ref[...], preferred_element_type=jnp.float32)
    _epilogue(y, xb_ref[...], w1_ref, b1_ref, w2_ref, b2_ref, o_ref, o16_ref)


@functools.partial(jax.jit, static_argnames=("bm",))
def _layer1(a_hat, x16, x_prev, w1, b1, w2, b2, bm=400):
    n, d = x_prev.shape
    m = a_hat.shape[0]
    return pl.pallas_call(
        _layer1_body,
        grid=(pl.cdiv(m, bm),),
        in_specs=[
            pl.BlockSpec((bm, n), lambda i: (i, 0)),
            pl.BlockSpec((n, d), lambda i: (0, 0)),
            pl.BlockSpec((bm, d), lambda i: (i, 0)),
            pl.BlockSpec((d, d), lambda i: (0, 0)),
            pl.BlockSpec((1, d), lambda i: (0, 0)),
            pl.BlockSpec((d, d), lambda i: (0, 0)),
            pl.BlockSpec((1, d), lambda i: (0, 0)),
        ],
        out_specs=[
            pl.BlockSpec((bm, d), lambda i: (i, 0)),
            pl.BlockSpec((bm, d), lambda i: (i, 0)),
            pl.BlockSpec((bm, n), lambda i: (i, 0)),
        ],
        out_shape=[
            jax.ShapeDtypeStruct((m, d), jnp.float32),
            jax.ShapeDtypeStruct((m, d), jnp.bfloat16),
            jax.ShapeDtypeStruct((m, n), jnp.bfloat16),
        ],
    )(a_hat, x16, x_prev, w1, b1, w2, b2)


@functools.partial(jax.jit, static_argnames=("bm",))
def _layer_bf16(a16, x16, x_prev, w1, b1, w2, b2, bm=624):
    n, d = x_prev.shape
    m = a16.shape[0]
    return pl.pallas_call(
        _layer_bf16_body,
        grid=(pl.cdiv(m, bm),),
        in_specs=[
            pl.BlockSpec((bm, n), lambda i: (i, 0)),
            pl.BlockSpec((n, d), lambda i: (0, 0)),
            pl.BlockSpec((bm, d), lambda i: (i, 0)),
            pl.BlockSpec((d, d), lambda i: (0, 0)),
            pl.BlockSpec((1, d), lambda i: (0, 0)),
            pl.BlockSpec((d, d), lambda i: (0, 0)),
            pl.BlockSpec((1, d), lambda i: (0, 0)),
        ],
        out_specs=[
            pl.BlockSpec((bm, d), lambda i: (i, 0)),
            pl.BlockSpec((bm, d), lambda i: (i, 0)),
        ],
        out_shape=[
            jax.ShapeDtypeStruct((m, d), jnp.float32),
            jax.ShapeDtypeStruct((m, d), jnp.bfloat16),
        ],
    )(a16, x16, x_prev, w1, b1, w2, b2)


def kernel(x, A_hat, embed, W11, b11, W12, b12, W21, b21, W22, b22, W31, b31,
           W32, b32):
    nb = x.shape[0]
    bw = (nb // 32) // 8 * 8
    rem = nb - 32 * bw
    x0 = _sc_gather(embed, x, bw, rem)
    x0_16 = x0.astype(jnp.bfloat16)
    b = [b.reshape(1, -1) for b in (b11, b12, b21, b22, b31, b32)]
    x1, x1_16, a16 = _layer1(A_hat, x0_16, x0, W11, b[0], W12, b[1])
    x2, x2_16 = _layer_bf16(a16, x1_16, x1, W21, b[2], W22, b[3])
    x3, _ = _layer_bf16(a16, x2_16, x2, W31, b[4], W32, b[5])
    return jnp.concatenate((x0, x1, x2, x3), axis=1)


# speedup vs baseline: 1.0962x; 1.0792x over previous
import functools

import jax
import jax.numpy as jnp
from jax import lax
from jax.experimental import pallas as pl
from jax.experimental.pallas import tpu as pltpu
from jax.experimental.pallas import tpu_sc as plsc


@functools.partial(jax.jit, static_argnames=("bw", "rem", "dout"))
def _sc_gather(embed, idx, bw, rem, dout):
    n, d = embed.shape
    mesh = plsc.VectorSubcoreMesh(core_axis_name="c", subcore_axis_name="s")
    scratch = [
        pltpu.VMEM((bw,), jnp.int32),
        pltpu.VMEM((bw, d), jnp.float32),
        pltpu.SemaphoreType.DMA,
    ]
    if rem:
        scratch += [
            pltpu.VMEM((rem,), jnp.int32),
            pltpu.VMEM((rem, d), jnp.float32),
        ]

    @functools.partial(
        pl.kernel, mesh=mesh,
        out_type=jax.ShapeDtypeStruct((idx.shape[0], dout), jnp.float32),
        scratch_types=scratch,
    )
    def body(table_hbm, idx_hbm, out_hbm, idx_v, rows_v, sem, *tail):
        wid = lax.axis_index("s") * 2 + lax.axis_index("c")
        base = wid * bw
        pltpu.sync_copy(idx_hbm.at[pl.ds(base, bw)], idx_v)
        pltpu.async_copy(table_hbm.at[idx_v], rows_v, sem).wait()
        pltpu.sync_copy(rows_v, out_hbm.at[pl.ds(base, bw), pl.ds(0, d)])
        if rem:
            idx2_v, rows2_v = tail

            @pl.when(wid == 0)
            def _():
                tbase = 32 * bw
                pltpu.sync_copy(idx_hbm.at[pl.ds(tbase, rem)], idx2_v)
                pltpu.async_copy(table_hbm.at[idx2_v], rows2_v, sem).wait()
                pltpu.sync_copy(rows2_v,
                                out_hbm.at[pl.ds(tbase, rem), pl.ds(0, d)])

    return body(embed, idx)


def _epilogue(y, xb, w1_ref, b1_ref, w2_ref, b2_ref, o_ref, o16_ref):
    s = y + xb
    p = y * xb
    t = jax.lax.dot_general(
        s, w1_ref[...], (((1,), (1,)), ((), ())),
        preferred_element_type=jnp.float32)
    t = t + jax.lax.dot_general(
        p, w2_ref[...], (((1,), (1,)), ((), ())),
        preferred_element_type=jnp.float32)
    t = t + b1_ref[...] + b2_ref[...]
    t = jnp.where(t >= 0, t, 0.2 * t)
    o_ref[...] = t
    o16_ref[...] = t.astype(jnp.bfloat16)


def _xb_rows(xf_ref, bm):
    i = pl.program_id(0)
    return xf_ref[pl.ds(i * bm, bm), :].astype(jnp.float32)


def _layer1_body(bm, a_ref, xf_ref, w1_ref, b1_ref, w2_ref, b2_ref, _buf_ref,
                 o_ref, o16_ref, a16_ref):
    a16 = a_ref[...].astype(jnp.bfloat16)
    a16_ref[...] = a16
    y = jnp.dot(a16, xf_ref[...], preferred_element_type=jnp.float32)
    _epilogue(y, _xb_rows(xf_ref, bm), w1_ref, b1_ref, w2_ref, b2_ref,
              o_ref, o16_ref)


def _layer_bf16_body(bm, a_ref, xf_ref, w1_ref, b1_ref, w2_ref, b2_ref,
                     _buf_ref, o_ref, o16_ref):
    y = jnp.dot(a_ref[...], xf_ref[...], preferred_element_type=jnp.float32)
    _epilogue(y, _xb_rows(xf_ref, bm), w1_ref, b1_ref, w2_ref, b2_ref,
              o_ref, o16_ref)


def _common_specs(bm, n, d, col):
    in_specs = [
        pl.BlockSpec((bm, n), lambda i: (i, 0)),
        pl.BlockSpec((n, d), lambda i: (0, 0)),
        pl.BlockSpec((d, d), lambda i: (0, 0)),
        pl.BlockSpec((1, d), lambda i: (0, 0)),
        pl.BlockSpec((d, d), lambda i: (0, 0)),
        pl.BlockSpec((1, d), lambda i: (0, 0)),
        pl.BlockSpec((8, d), lambda i: (0, 0)),
    ]
    buf_spec = pl.BlockSpec((bm, d), lambda i: (i, col))
    x16_spec = pl.BlockSpec((bm, d), lambda i: (i, 0))
    return in_specs, buf_spec, x16_spec


@functools.partial(jax.jit, static_argnames=("bm", "col"))
def _layer1(a_hat, x16, buf, w1, b1, w2, b2, bm, col):
    n, d = x16.shape
    m = a_hat.shape[0]
    in_specs, buf_spec, x16_spec = _common_specs(bm, n, d, col)
    return pl.pallas_call(
        functools.partial(_layer1_body, bm),
        grid=(m // bm,),
        in_specs=in_specs,
        out_specs=[
            buf_spec,
            x16_spec,
            pl.BlockSpec((bm, n), lambda i: (i, 0)),
        ],
        out_shape=[
            jax.ShapeDtypeStruct(buf.shape, jnp.float32),
            jax.ShapeDtypeStruct((m, d), jnp.bfloat16),
            jax.ShapeDtypeStruct((m, n), jnp.bfloat16),
        ],
        input_output_aliases={6: 0},
    )(a_hat, x16, w1, b1, w2, b2, buf)


@functools.partial(jax.jit, static_argnames=("bm", "col"))
def _layer_bf16(a16, x16, buf, w1, b1, w2, b2, bm, col):
    n, d = x16.shape
    m = a16.shape[0]
    in_specs, buf_spec, x16_spec = _common_specs(bm, n, d, col)
    return pl.pallas_call(
        functools.partial(_layer_bf16_body, bm),
        grid=(m // bm,),
        in_specs=in_specs,
        out_specs=[buf_spec, x16_spec],
        out_shape=[
            jax.ShapeDtypeStruct(buf.shape, jnp.float32),
            jax.ShapeDtypeStruct((m, d), jnp.bfloat16),
        ],
        input_output_aliases={6: 0},
    )(a16, x16, w1, b1, w2, b2, buf)


def _run_layers(buf, A_hat, W11, b11, W12, b12, W21, b21, W22, b22, W31, b31,
                W32, b32):
    n = A_hat.shape[0]
    bm1 = 400 if n % 400 == 0 else n
    bm2 = 1000 if n % 1000 == 0 else n
    x0_16 = buf[:, :128].astype(jnp.bfloat16)
    b = [bb.reshape(1, -1) for bb in (b11, b12, b21, b22, b31, b32)]
    buf, x1_16, a16 = _layer1(A_hat, x0_16, buf, W11, b[0], W12, b[1],
                              bm1, 1)
    buf, x2_16 = _layer_bf16(a16, x1_16, buf, W21, b[2], W22, b[3], bm2, 2)
    buf, _ = _layer_bf16(a16, x2_16, buf, W31, b[4], W32, b[5], bm2, 3)
    return buf


def kernel(x, A_hat, embed, W11, b11, W12, b12, W21, b21, W22, b22, W31, b31,
           W32, b32):
    nb = x.shape[0]
    d = embed.shape[1]
    bw = (nb // 32) // 8 * 8
    rem = nb - 32 * bw
    buf = _sc_gather(embed, x, bw, rem, 4 * d)
    return _run_layers(buf, A_hat, W11, b11, W12, b12, W21, b21, W22, b22,
                       W31, b31, W32, b32)
